# baseline (speedup 1.0000x reference)
import functools

import jax
import jax.numpy as jnp
from jax import lax
from jax.experimental import pallas as pl
from jax.experimental.pallas import tpu as pltpu
from jax.experimental.pallas import tpu_sc as plsc

N = 10000
N_PAD = 10240
HF = 32
NC = 2
NS = 16
NW = NC * NS
CHUNK = 128
NCHUNK = 80
E_PAD = NW * NCHUNK * CHUNK
ROWS_PER_TILE = N_PAD // NS
NBUF = 4


def _edge_body(p_hbm, src_hbm, dst_hbm, out_hbm,
               acc_sh, idx_s, idx_d, rows, stg, semg, sems):
    cid = lax.axis_index("c")
    sid = lax.axis_index("s")
    wid = cid * NS + sid

    z16 = jnp.zeros((16,), jnp.float32)

    @pl.loop(0, ROWS_PER_TILE)
    def _(i):
        stg[i, pl.ds(0, 16)] = z16
        stg[i, pl.ds(16, 16)] = z16

    base = sid * ROWS_PER_TILE
    pltpu.sync_copy(stg, acc_sh.at[pl.ds(base, ROWS_PER_TILE)])

    pltpu.sync_copy(src_hbm.at[wid], idx_s)
    pltpu.sync_copy(dst_hbm.at[wid], idx_d)

    plsc.subcore_barrier()

    for b in range(NBUF):
        pltpu.async_copy(p_hbm.at[idx_s.at[b]], rows.at[b], semg.at[b])

    @pl.loop(0, NCHUNK, step=NBUF)
    def _(j):
        for b in range(NBUF):
            c = j + b
            pltpu.make_async_copy(p_hbm.at[idx_s.at[c]], rows.at[b],
                                  semg.at[b]).wait()
            pltpu.async_copy(rows.at[b], acc_sh.at[idx_d.at[c]],
                             sems.at[b], add=True)
            bp = (b - 1) % NBUF
            cp = c - 1

            @pl.when((cp >= 0) & (cp + NBUF < NCHUNK))
            def _():
                pltpu.make_async_copy(rows.at[bp], acc_sh.at[idx_d.at[cp]],
                                      sems.at[bp]).wait()
                pltpu.async_copy(p_hbm.at[idx_s.at[cp + NBUF]], rows.at[bp],
                                 semg.at[bp])

    for b in range(NBUF):
        c = NCHUNK - NBUF + b
        pltpu.make_async_copy(rows.at[b], acc_sh.at[idx_d.at[c]],
                              sems.at[b]).wait()

    plsc.subcore_barrier()

    pltpu.sync_copy(acc_sh.at[pl.ds(base, ROWS_PER_TILE)], stg)
    pltpu.sync_copy(stg, out_hbm.at[cid, pl.ds(base, ROWS_PER_TILE)])


_edge_pass = pl.kernel(
    _edge_body,
    out_type=jax.ShapeDtypeStruct((NC, N_PAD, HF), jnp.float32),
    mesh=plsc.VectorSubcoreMesh(core_axis_name="c", subcore_axis_name="s"),
    scratch_types=[
        pltpu.VMEM_SHARED((N_PAD, HF), jnp.float32),
        pltpu.VMEM((NCHUNK, CHUNK), jnp.int32),
        pltpu.VMEM((NCHUNK, CHUNK), jnp.int32),
        pltpu.VMEM((NBUF, CHUNK, HF), jnp.float32),
        pltpu.VMEM((ROWS_PER_TILE, HF), jnp.float32),
        pltpu.SemaphoreType.DMA((NBUF,)),
        pltpu.SemaphoreType.DMA((NBUF,)),
    ],
    compiler_params=pltpu.CompilerParams(use_tc_tiling_on_sc=False),
)


def _prep_body(x_ref, w_ref, b_ref, degp_ref, x0_ref, p0_ref, dv_ref):
    h = jnp.dot(x_ref[...], w_ref[...], preferred_element_type=jnp.float32)
    h = jnp.maximum(h + b_ref[...], 0.0)
    mn = jnp.min(h, axis=1, keepdims=True)
    mx = jnp.max(h, axis=1, keepdims=True)
    x0 = 2.0 * (h - mn) / (mx - mn + 1e-08) - 1.0
    deg = degp_ref[0, :, 0:1] + degp_ref[1, :, 0:1] + 1.0
    dv = jnp.broadcast_to(lax.rsqrt(deg), x0.shape)
    x0_ref[...] = x0
    p0_ref[...] = x0 * dv
    dv_ref[...] = dv


def _layer_body(acc_ref, p_ref, dv_ref, w_ref, b_ref, x_ref, po_ref):
    dv = dv_ref[...]
    t = (acc_ref[0] + acc_ref[1] + p_ref[...]) * dv
    xi = jnp.dot(t, w_ref[...], preferred_element_type=jnp.float32) + b_ref[...]
    x_ref[...] = xi
    po_ref[...] = xi * dv


def _final_body(*refs):
    xs = refs[:32]
    w_ref, b_ref, out_ref = refs[32], refs[33], refs[34]

    def mynorm(t):
        mn = jnp.min(t, axis=1, keepdims=True)
        mx = jnp.max(t, axis=1, keepdims=True)
        return 2.0 * (t - mn) / (mx - mn + 1e-08) - 1.0

    vals = [x[...] for x in xs]
    norms = [None, None] + [mynorm(v) for v in vals[2:]]
    norms[0] = mynorm(vals[0])
    norms[1] = mynorm(vals[1])
    parts = [vals[0], vals[1]]
    for i in range(2, 32):
        parts.append(norms[i] - norms[i - 2])
    parts[16] = vals[16]
    xx = jnp.concatenate(parts, axis=1)
    out_ref[...] = (
        jnp.dot(xx, w_ref[...], preferred_element_type=jnp.float32)
        + b_ref[...])


_R = 1024
_GRID = N_PAD // _R


def _rows_spec(width):
    return pl.BlockSpec((_R, width), lambda i: (i, 0))


def _whole_spec(shape):
    nd = len(shape)
    return pl.BlockSpec(shape, lambda i: (0,) * nd)


_prep_call = pl.pallas_call(
    _prep_body,
    grid=(_GRID,),
    in_specs=[
        _rows_spec(128),
        _whole_spec((128, HF)),
        _whole_spec((1, HF)),
        pl.BlockSpec((NC, _R, HF), lambda i: (0, i, 0)),
    ],
    out_specs=[_rows_spec(HF)] * 3,
    out_shape=[jax.ShapeDtypeStruct((N_PAD, HF), jnp.float32)] * 3,
)

_layer_call = pl.pallas_call(
    _layer_body,
    grid=(_GRID,),
    in_specs=[
        pl.BlockSpec((NC, _R, HF), lambda i: (0, i, 0)),
        _rows_spec(HF),
        _rows_spec(HF),
        _whole_spec((HF, HF)),
        _whole_spec((1, HF)),
    ],
    out_specs=[_rows_spec(HF)] * 2,
    out_shape=[jax.ShapeDtypeStruct((N_PAD, HF), jnp.float32)] * 2,
)

_final_call = pl.pallas_call(
    _final_body,
    grid=(_GRID,),
    in_specs=(
        [_rows_spec(HF)] * 32
        + [_whole_spec((32 * HF, 128)), _whole_spec((1, 128))]
    ),
    out_specs=_rows_spec(128),
    out_shape=jax.ShapeDtypeStruct((N_PAD, 128), jnp.float32),
)


@jax.jit
def kernel(x, edge_index, W_fc1, b_fc1, Wc, bc, W_out, b_out):
    src = edge_index[0].astype(jnp.int32)
    dst = edge_index[1].astype(jnp.int32)
    e = src.shape[0]
    pad = E_PAD - e
    src3 = jnp.concatenate(
        [src, jnp.zeros((pad,), jnp.int32)]).reshape(NW, NCHUNK, CHUNK)
    dst3 = jnp.concatenate(
        [dst, jnp.full((pad,), N, jnp.int32)]).reshape(NW, NCHUNK, CHUNK)

    xp = jnp.zeros((N_PAD, x.shape[1]), x.dtype).at[:N].set(x)
    ones = jnp.ones((N_PAD, HF), jnp.float32)

    degp = _edge_pass(ones, dst3, dst3)
    x0, p, dv = _prep_call(xp, W_fc1, b_fc1.reshape(1, HF), degp)

    xs = [x0]
    for i in range(31):
        accp = _edge_pass(p, src3, dst3)
        xi, p = _layer_call(accp, p, dv, Wc[i], bc[i].reshape(1, HF))
        xs.append(xi)

    out = _final_call(*xs, W_out, b_out.reshape(1, 128))
    return out[:N]


# speedup vs baseline: 2.2431x; 2.2431x over previous
import functools

import jax
import jax.numpy as jnp
from jax import lax
from jax.experimental import pallas as pl
from jax.experimental.pallas import tpu as pltpu
from jax.experimental.pallas import tpu_sc as plsc

N = 10000
N_PAD = 10240
HF = 32
NC = 2
NS = 16
NW = NC * NS
CHUNK = 128
NCHUNK = 80
E_PAD = NW * NCHUNK * CHUNK
ROWS_PER_TILE = N_PAD // NS
NBUF = 4


def _edge_body(p_hbm, src_hbm, dst_hbm, out_hbm,
               acc_sh, p_sh, idx_s, idx_d, rows, stg, semg, sems):
    cid = lax.axis_index("c")
    sid = lax.axis_index("s")
    wid = cid * NS + sid

    base = sid * ROWS_PER_TILE
    pltpu.sync_copy(p_hbm.at[pl.ds(base, ROWS_PER_TILE)],
                    p_sh.at[pl.ds(base, ROWS_PER_TILE)])

    z16 = jnp.zeros((16,), jnp.float32)

    @pl.loop(0, ROWS_PER_TILE)
    def _(i):
        stg[i, pl.ds(0, 16)] = z16
        stg[i, pl.ds(16, 16)] = z16

    pltpu.sync_copy(stg, acc_sh.at[pl.ds(base, ROWS_PER_TILE)])

    pltpu.sync_copy(src_hbm.at[wid], idx_s)
    pltpu.sync_copy(dst_hbm.at[wid], idx_d)

    plsc.subcore_barrier()

    for b in range(NBUF):
        pltpu.async_copy(p_sh.at[idx_s.at[b]], rows.at[b], semg.at[b])

    @pl.loop(0, NCHUNK, step=NBUF)
    def _(j):
        for b in range(NBUF):
            c = j + b
            pltpu.make_async_copy(p_sh.at[idx_s.at[c]], rows.at[b],
                                  semg.at[b]).wait()
            pltpu.async_copy(rows.at[b], acc_sh.at[idx_d.at[c]],
                             sems.at[b], add=True)
            bp = (b - 1) % NBUF
            cp = c - 1

            @pl.when((cp >= 0) & (cp + NBUF < NCHUNK))
            def _():
                pltpu.make_async_copy(rows.at[bp], acc_sh.at[idx_d.at[cp]],
                                      sems.at[bp]).wait()
                pltpu.async_copy(p_sh.at[idx_s.at[cp + NBUF]], rows.at[bp],
                                 semg.at[bp])

    for b in range(NBUF):
        c = NCHUNK - NBUF + b
        pltpu.make_async_copy(rows.at[b], acc_sh.at[idx_d.at[c]],
                              sems.at[b]).wait()

    plsc.subcore_barrier()

    pltpu.sync_copy(acc_sh.at[pl.ds(base, ROWS_PER_TILE)], stg)
    pltpu.sync_copy(stg, out_hbm.at[cid, pl.ds(base, ROWS_PER_TILE)])


_edge_pass = pl.kernel(
    _edge_body,
    out_type=jax.ShapeDtypeStruct((NC, N_PAD, HF), jnp.float32),
    mesh=plsc.VectorSubcoreMesh(core_axis_name="c", subcore_axis_name="s"),
    scratch_types=[
        pltpu.VMEM_SHARED((N_PAD, HF), jnp.float32),
        pltpu.VMEM_SHARED((N_PAD, HF), jnp.float32),
        pltpu.VMEM((NCHUNK, CHUNK), jnp.int32),
        pltpu.VMEM((NCHUNK, CHUNK), jnp.int32),
        pltpu.VMEM((NBUF, CHUNK, HF), jnp.float32),
        pltpu.VMEM((ROWS_PER_TILE, HF), jnp.float32),
        pltpu.SemaphoreType.DMA((NBUF,)),
        pltpu.SemaphoreType.DMA((NBUF,)),
    ],
    compiler_params=pltpu.CompilerParams(use_tc_tiling_on_sc=False),
)


def _prep_body(x_ref, w_ref, b_ref, degp_ref, x0_ref, p0_ref, dv_ref):
    h = jnp.dot(x_ref[...], w_ref[...], preferred_element_type=jnp.float32)
    h = jnp.maximum(h + b_ref[...], 0.0)
    mn = jnp.min(h, axis=1, keepdims=True)
    mx = jnp.max(h, axis=1, keepdims=True)
    x0 = 2.0 * (h - mn) / (mx - mn + 1e-08) - 1.0
    deg = degp_ref[0, :, 0:1] + degp_ref[1, :, 0:1] + 1.0
    dv = jnp.broadcast_to(lax.rsqrt(deg), x0.shape)
    x0_ref[...] = x0
    p0_ref[...] = x0 * dv
    dv_ref[...] = dv


def _layer_body(acc_ref, p_ref, dv_ref, w_ref, b_ref, x_ref, po_ref):
    dv = dv_ref[...]
    t = (acc_ref[0] + acc_ref[1] + p_ref[...]) * dv
    xi = jnp.dot(t, w_ref[...], preferred_element_type=jnp.float32) + b_ref[...]
    x_ref[...] = xi
    po_ref[...] = xi * dv


def _final_body(*refs):
    xs = refs[:32]
    w_ref, b_ref, out_ref = refs[32], refs[33], refs[34]

    def mynorm(t):
        mn = jnp.min(t, axis=1, keepdims=True)
        mx = jnp.max(t, axis=1, keepdims=True)
        return 2.0 * (t - mn) / (mx - mn + 1e-08) - 1.0

    vals = [x[...] for x in xs]
    norms = [None, None] + [mynorm(v) for v in vals[2:]]
    norms[0] = mynorm(vals[0])
    norms[1] = mynorm(vals[1])
    parts = [vals[0], vals[1]]
    for i in range(2, 32):
        parts.append(norms[i] - norms[i - 2])
    parts[16] = vals[16]
    xx = jnp.concatenate(parts, axis=1)
    out_ref[...] = (
        jnp.dot(xx, w_ref[...], preferred_element_type=jnp.float32)
        + b_ref[...])


_R = 1024
_GRID = N_PAD // _R


def _rows_spec(width):
    return pl.BlockSpec((_R, width), lambda i: (i, 0))


def _whole_spec(shape):
    nd = len(shape)
    return pl.BlockSpec(shape, lambda i: (0,) * nd)


_prep_call = pl.pallas_call(
    _prep_body,
    grid=(_GRID,),
    in_specs=[
        _rows_spec(128),
        _whole_spec((128, HF)),
        _whole_spec((1, HF)),
        pl.BlockSpec((NC, _R, HF), lambda i: (0, i, 0)),
    ],
    out_specs=[_rows_spec(HF)] * 3,
    out_shape=[jax.ShapeDtypeStruct((N_PAD, HF), jnp.float32)] * 3,
)

_layer_call = pl.pallas_call(
    _layer_body,
    grid=(_GRID,),
    in_specs=[
        pl.BlockSpec((NC, _R, HF), lambda i: (0, i, 0)),
        _rows_spec(HF),
        _rows_spec(HF),
        _whole_spec((HF, HF)),
        _whole_spec((1, HF)),
    ],
    out_specs=[_rows_spec(HF)] * 2,
    out_shape=[jax.ShapeDtypeStruct((N_PAD, HF), jnp.float32)] * 2,
)

_final_call = pl.pallas_call(
    _final_body,
    grid=(_GRID,),
    in_specs=(
        [_rows_spec(HF)] * 32
        + [_whole_spec((32 * HF, 128)), _whole_spec((1, 128))]
    ),
    out_specs=_rows_spec(128),
    out_shape=jax.ShapeDtypeStruct((N_PAD, 128), jnp.float32),
)


@jax.jit
def kernel(x, edge_index, W_fc1, b_fc1, Wc, bc, W_out, b_out):
    src = edge_index[0].astype(jnp.int32)
    dst = edge_index[1].astype(jnp.int32)
    e = src.shape[0]
    pad = E_PAD - e
    src3 = jnp.concatenate(
        [src, jnp.zeros((pad,), jnp.int32)]).reshape(NW, NCHUNK, CHUNK)
    dst3 = jnp.concatenate(
        [dst, jnp.full((pad,), N, jnp.int32)]).reshape(NW, NCHUNK, CHUNK)

    xp = jnp.zeros((N_PAD, x.shape[1]), x.dtype).at[:N].set(x)
    ones = jnp.ones((N_PAD, HF), jnp.float32)

    degp = _edge_pass(ones, dst3, dst3)
    x0, p, dv = _prep_call(xp, W_fc1, b_fc1.reshape(1, HF), degp)

    xs = [x0]
    for i in range(31):
        accp = _edge_pass(p, src3, dst3)
        xi, p = _layer_call(accp, p, dv, Wc[i], bc[i].reshape(1, HF))
        xs.append(xi)

    out = _final_call(*xs, W_out, b_out.reshape(1, 128))
    return out[:N]


# speedup vs baseline: 2.5037x; 1.1162x over previous
import functools

import jax
import jax.numpy as jnp
from jax import lax
from jax.experimental import pallas as pl
from jax.experimental.pallas import tpu as pltpu
from jax.experimental.pallas import tpu_sc as plsc

N = 10000
N_PAD = 10240
HF = 32
NC = 2
NS = 16
NW = NC * NS
CHUNK = 128
NCHUNK = 80
E_PAD = NW * NCHUNK * CHUNK
ROWS_PER_TILE = N_PAD // NS
NBUF = 4


def _edge_body(p_hbm, src_hbm, dst_hbm, zeros_hbm, out_hbm,
               acc_sh, p_sh, idx_s, idx_d, rows, stg, semstg, semg, sems):
    cid = lax.axis_index("c")
    sid = lax.axis_index("s")
    wid = cid * NS + sid
    base = sid * ROWS_PER_TILE
    stripe = pl.ds(base, ROWS_PER_TILE)

    pltpu.async_copy(p_hbm.at[stripe], p_sh.at[stripe], semstg)

    @pl.when(cid == 0)
    def _():
        pltpu.async_copy(p_hbm.at[stripe], acc_sh.at[stripe], semstg)

    @pl.when(cid != 0)
    def _():
        pltpu.async_copy(zeros_hbm.at[stripe], acc_sh.at[stripe], semstg)

    pltpu.async_copy(src_hbm.at[wid], idx_s, semstg)
    pltpu.async_copy(dst_hbm.at[wid], idx_d, semstg)

    pltpu.make_async_copy(p_hbm.at[stripe], p_sh.at[stripe], semstg).wait()
    pltpu.make_async_copy(zeros_hbm.at[stripe], acc_sh.at[stripe],
                          semstg).wait()
    pltpu.make_async_copy(src_hbm.at[wid], idx_s, semstg).wait()
    pltpu.make_async_copy(dst_hbm.at[wid], idx_d, semstg).wait()

    plsc.subcore_barrier()

    for b in range(NBUF):
        pltpu.async_copy(p_sh.at[idx_s.at[b]], rows.at[b], semg.at[b])

    @pl.loop(0, NCHUNK, step=NBUF)
    def _(j):
        for b in range(NBUF):
            c = j + b
            pltpu.make_async_copy(p_sh.at[idx_s.at[c]], rows.at[b],
                                  semg.at[b]).wait()
            pltpu.async_copy(rows.at[b], acc_sh.at[idx_d.at[c]],
                             sems.at[b], add=True)
            bp = (b - 1) % NBUF
            cp = c - 1

            @pl.when((cp >= 0) & (cp + NBUF < NCHUNK))
            def _():
                pltpu.make_async_copy(rows.at[bp], acc_sh.at[idx_d.at[cp]],
                                      sems.at[bp]).wait()
                pltpu.async_copy(p_sh.at[idx_s.at[cp + NBUF]], rows.at[bp],
                                 semg.at[bp])

    for b in range(NBUF):
        c = NCHUNK - NBUF + b
        pltpu.make_async_copy(rows.at[b], acc_sh.at[idx_d.at[c]],
                              sems.at[b]).wait()

    plsc.subcore_barrier()

    pltpu.sync_copy(acc_sh.at[stripe], stg)
    pltpu.sync_copy(stg, out_hbm.at[cid, stripe])


_edge_pass = pl.kernel(
    _edge_body,
    out_type=jax.ShapeDtypeStruct((NC, N_PAD, HF), jnp.float32),
    mesh=plsc.VectorSubcoreMesh(core_axis_name="c", subcore_axis_name="s"),
    scratch_types=[
        pltpu.VMEM_SHARED((N_PAD, HF), jnp.float32),
        pltpu.VMEM_SHARED((N_PAD, HF), jnp.float32),
        pltpu.VMEM((NCHUNK, CHUNK), jnp.int32),
        pltpu.VMEM((NCHUNK, CHUNK), jnp.int32),
        pltpu.VMEM((NBUF, CHUNK, HF), jnp.float32),
        pltpu.VMEM((ROWS_PER_TILE, HF), jnp.float32),
        pltpu.SemaphoreType.DMA,
        pltpu.SemaphoreType.DMA((NBUF,)),
        pltpu.SemaphoreType.DMA((NBUF,)),
    ],
    compiler_params=pltpu.CompilerParams(use_tc_tiling_on_sc=False),
)

DEGW = 16


def _deg_body(dst_hbm, zeros_hbm, out_hbm, acc_sh, idx_d, ones, stg, semstg,
              sems):
    cid = lax.axis_index("c")
    sid = lax.axis_index("s")
    wid = cid * NS + sid
    base = sid * ROWS_PER_TILE
    stripe = pl.ds(base, ROWS_PER_TILE)

    pltpu.async_copy(zeros_hbm.at[stripe], acc_sh.at[stripe], semstg)
    pltpu.async_copy(dst_hbm.at[wid], idx_d, semstg)

    one16 = jnp.ones((16,), jnp.float32)

    @pl.loop(0, CHUNK)
    def _(i):
        ones[i, pl.ds(0, DEGW)] = one16

    pltpu.make_async_copy(zeros_hbm.at[stripe], acc_sh.at[stripe],
                          semstg).wait()
    pltpu.make_async_copy(dst_hbm.at[wid], idx_d, semstg).wait()

    plsc.subcore_barrier()

    for g in range(0, NCHUNK, 16):
        for c in range(g, g + 16):
            pltpu.async_copy(ones, acc_sh.at[idx_d.at[c]], sems, add=True)
        for c in range(g, g + 16):
            pltpu.make_async_copy(ones, acc_sh.at[idx_d.at[c]], sems).wait()

    plsc.subcore_barrier()

    pltpu.sync_copy(acc_sh.at[stripe], stg)
    pltpu.sync_copy(stg, out_hbm.at[cid, stripe])


_deg_pass = pl.kernel(
    _deg_body,
    out_type=jax.ShapeDtypeStruct((NC, N_PAD, DEGW), jnp.float32),
    mesh=plsc.VectorSubcoreMesh(core_axis_name="c", subcore_axis_name="s"),
    scratch_types=[
        pltpu.VMEM_SHARED((N_PAD, DEGW), jnp.float32),
        pltpu.VMEM((NCHUNK, CHUNK), jnp.int32),
        pltpu.VMEM((CHUNK, DEGW), jnp.float32),
        pltpu.VMEM((ROWS_PER_TILE, DEGW), jnp.float32),
        pltpu.SemaphoreType.DMA,
        pltpu.SemaphoreType.DMA,
    ],
    compiler_params=pltpu.CompilerParams(use_tc_tiling_on_sc=False),
)


def _prep_body(x_ref, w_ref, b_ref, degp_ref, x0_ref, p0_ref, dv_ref):
    h = jnp.dot(x_ref[...], w_ref[...], preferred_element_type=jnp.float32)
    h = jnp.maximum(h + b_ref[...], 0.0)
    mn = jnp.min(h, axis=1, keepdims=True)
    mx = jnp.max(h, axis=1, keepdims=True)
    x0 = 2.0 * (h - mn) / (mx - mn + 1e-08) - 1.0
    deg = degp_ref[0, :, 0:1] + degp_ref[1, :, 0:1] + 1.0
    dv = jnp.broadcast_to(lax.rsqrt(deg), x0.shape)
    x0_ref[...] = x0
    p0_ref[...] = x0 * dv
    dv_ref[...] = dv


def _layer_body(acc_ref, dv_ref, w_ref, b_ref, x_ref, po_ref):
    dv = dv_ref[...]
    t = (acc_ref[0] + acc_ref[1]) * dv
    xi = jnp.dot(t, w_ref[...], preferred_element_type=jnp.float32) + b_ref[...]
    x_ref[...] = xi
    po_ref[...] = xi * dv


def _final_body(*refs):
    xs = refs[:32]
    w_ref, b_ref, out_ref = refs[32], refs[33], refs[34]

    def mynorm(t):
        mn = jnp.min(t, axis=1, keepdims=True)
        mx = jnp.max(t, axis=1, keepdims=True)
        return 2.0 * (t - mn) / (mx - mn + 1e-08) - 1.0

    vals = [x[...] for x in xs]
    norms = [mynorm(v) for v in vals]
    parts = [vals[0], vals[1]]
    for i in range(2, 32):
        parts.append(norms[i] - norms[i - 2])
    parts[16] = vals[16]
    xx = jnp.concatenate(parts, axis=1)
    out_ref[...] = (
        jnp.dot(xx, w_ref[...], preferred_element_type=jnp.float32)
        + b_ref[...])


def _whole_spec(shape):
    nd = len(shape)
    return pl.BlockSpec(shape, lambda *_: (0,) * nd)


_prep_call = pl.pallas_call(
    _prep_body,
    in_specs=[
        _whole_spec((N_PAD, 128)),
        _whole_spec((128, HF)),
        _whole_spec((1, HF)),
        _whole_spec((NC, N_PAD, DEGW)),
    ],
    out_specs=[_whole_spec((N_PAD, HF))] * 3,
    out_shape=[jax.ShapeDtypeStruct((N_PAD, HF), jnp.float32)] * 3,
)

_layer_call = pl.pallas_call(
    _layer_body,
    in_specs=[
        _whole_spec((NC, N_PAD, HF)),
        _whole_spec((N_PAD, HF)),
        _whole_spec((HF, HF)),
        _whole_spec((1, HF)),
    ],
    out_specs=[_whole_spec((N_PAD, HF))] * 2,
    out_shape=[jax.ShapeDtypeStruct((N_PAD, HF), jnp.float32)] * 2,
)

_RF = 1024
_final_call = pl.pallas_call(
    _final_body,
    grid=(N_PAD // _RF,),
    in_specs=(
        [pl.BlockSpec((_RF, HF), lambda i: (i, 0))] * 32
        + [_whole_spec((32 * HF, 128)), _whole_spec((1, 128))]
    ),
    out_specs=pl.BlockSpec((_RF, 128), lambda i: (i, 0)),
    out_shape=jax.ShapeDtypeStruct((N_PAD, 128), jnp.float32),
)


@jax.jit
def kernel(x, edge_index, W_fc1, b_fc1, Wc, bc, W_out, b_out):
    src = edge_index[0].astype(jnp.int32)
    dst = edge_index[1].astype(jnp.int32)
    e = src.shape[0]
    pad = E_PAD - e
    src3 = jnp.concatenate(
        [src, jnp.zeros((pad,), jnp.int32)]).reshape(NW, NCHUNK, CHUNK)
    dst3 = jnp.concatenate(
        [dst, jnp.full((pad,), N, jnp.int32)]).reshape(NW, NCHUNK, CHUNK)

    xp = jnp.zeros((N_PAD, x.shape[1]), x.dtype).at[:N].set(x)
    zeros32 = jnp.zeros((N_PAD, HF), jnp.float32)
    zeros16 = jnp.zeros((N_PAD, DEGW), jnp.float32)

    degp = _deg_pass(dst3, zeros16)
    x0, p, dv = _prep_call(xp, W_fc1, b_fc1.reshape(1, HF), degp)

    xs = [x0]
    for i in range(31):
        accp = _edge_pass(p, src3, dst3, zeros32)
        xi, p = _layer_call(accp, dv, Wc[i], bc[i].reshape(1, HF))
        xs.append(xi)

    out = _final_call(*xs, W_out, b_out.reshape(1, 128))
    return out[:N]


# speedup vs baseline: 2.5072x; 1.0014x over previous
import functools

import jax
import jax.numpy as jnp
from jax import lax
from jax.experimental import pallas as pl
from jax.experimental.pallas import tpu as pltpu
from jax.experimental.pallas import tpu_sc as plsc

N = 10000
N_PAD = 10240
HF = 32
NC = 2
NS = 16
NW = NC * NS
CHUNK = 128
NCHUNK = 80
E_PAD = NW * NCHUNK * CHUNK
ROWS_PER_TILE = N_PAD // NS
NBUF = 4


def _edge_body(p_hbm, src_hbm, dst_hbm, zeros_hbm, out_hbm,
               acc_sh, p_sh, idx_s, idx_d, rows, semstg, semg, sems):
    cid = lax.axis_index("c")
    sid = lax.axis_index("s")
    wid = cid * NS + sid
    base = sid * ROWS_PER_TILE
    stripe = pl.ds(base, ROWS_PER_TILE)

    pltpu.async_copy(p_hbm.at[stripe], p_sh.at[stripe], semstg)

    @pl.when(cid == 0)
    def _():
        pltpu.async_copy(p_hbm.at[stripe], acc_sh.at[stripe], semstg)

    @pl.when(cid != 0)
    def _():
        pltpu.async_copy(zeros_hbm.at[stripe], acc_sh.at[stripe], semstg)

    pltpu.async_copy(src_hbm.at[wid], idx_s, semstg)
    pltpu.async_copy(dst_hbm.at[wid], idx_d, semstg)

    pltpu.make_async_copy(p_hbm.at[stripe], p_sh.at[stripe], semstg).wait()
    pltpu.make_async_copy(zeros_hbm.at[stripe], acc_sh.at[stripe],
                          semstg).wait()
    pltpu.make_async_copy(src_hbm.at[wid], idx_s, semstg).wait()
    pltpu.make_async_copy(dst_hbm.at[wid], idx_d, semstg).wait()

    plsc.subcore_barrier()

    for b in range(NBUF):
        pltpu.async_copy(p_sh.at[idx_s.at[b]], rows.at[b], semg.at[b])

    @pl.loop(0, NCHUNK, step=NBUF)
    def _(j):
        for b in range(NBUF):
            c = j + b
            pltpu.make_async_copy(p_sh.at[idx_s.at[c]], rows.at[b],
                                  semg.at[b]).wait()
            pltpu.async_copy(rows.at[b], acc_sh.at[idx_d.at[c]],
                             sems.at[b], add=True)
            bp = (b - 1) % NBUF
            cp = c - 1

            @pl.when((cp >= 0) & (cp + NBUF < NCHUNK))
            def _():
                pltpu.make_async_copy(rows.at[bp], acc_sh.at[idx_d.at[cp]],
                                      sems.at[bp]).wait()
                pltpu.async_copy(p_sh.at[idx_s.at[cp + NBUF]], rows.at[bp],
                                 semg.at[bp])

    for b in range(NBUF):
        c = NCHUNK - NBUF + b
        pltpu.make_async_copy(rows.at[b], acc_sh.at[idx_d.at[c]],
                              sems.at[b]).wait()

    plsc.subcore_barrier()

    pltpu.sync_copy(acc_sh.at[stripe], out_hbm.at[cid, stripe])


_edge_pass = pl.kernel(
    _edge_body,
    out_type=jax.ShapeDtypeStruct((NC, N_PAD, HF), jnp.float32),
    mesh=plsc.VectorSubcoreMesh(core_axis_name="c", subcore_axis_name="s"),
    scratch_types=[
        pltpu.VMEM_SHARED((N_PAD, HF), jnp.float32),
        pltpu.VMEM_SHARED((N_PAD, HF), jnp.float32),
        pltpu.VMEM((NCHUNK, CHUNK), jnp.int32),
        pltpu.VMEM((NCHUNK, CHUNK), jnp.int32),
        pltpu.VMEM((NBUF, CHUNK, HF), jnp.float32),
        pltpu.SemaphoreType.DMA,
        pltpu.SemaphoreType.DMA((NBUF,)),
        pltpu.SemaphoreType.DMA((NBUF,)),
    ],
    compiler_params=pltpu.CompilerParams(use_tc_tiling_on_sc=False),
)

DEGW = 16


def _deg_body(dst_hbm, zeros_hbm, out_hbm, acc_sh, idx_d, ones, semstg,
              sems):
    cid = lax.axis_index("c")
    sid = lax.axis_index("s")
    wid = cid * NS + sid
    base = sid * ROWS_PER_TILE
    stripe = pl.ds(base, ROWS_PER_TILE)

    pltpu.async_copy(zeros_hbm.at[stripe], acc_sh.at[stripe], semstg)
    pltpu.async_copy(dst_hbm.at[wid], idx_d, semstg)

    one16 = jnp.ones((16,), jnp.float32)

    @pl.loop(0, CHUNK)
    def _(i):
        ones[i, pl.ds(0, DEGW)] = one16

    pltpu.make_async_copy(zeros_hbm.at[stripe], acc_sh.at[stripe],
                          semstg).wait()
    pltpu.make_async_copy(dst_hbm.at[wid], idx_d, semstg).wait()

    plsc.subcore_barrier()

    for g in range(0, NCHUNK, 16):
        for c in range(g, g + 16):
            pltpu.async_copy(ones, acc_sh.at[idx_d.at[c]], sems, add=True)
        for c in range(g, g + 16):
            pltpu.make_async_copy(ones, acc_sh.at[idx_d.at[c]], sems).wait()

    plsc.subcore_barrier()

    pltpu.sync_copy(acc_sh.at[stripe], out_hbm.at[cid, stripe])


_deg_pass = pl.kernel(
    _deg_body,
    out_type=jax.ShapeDtypeStruct((NC, N_PAD, DEGW), jnp.float32),
    mesh=plsc.VectorSubcoreMesh(core_axis_name="c", subcore_axis_name="s"),
    scratch_types=[
        pltpu.VMEM_SHARED((N_PAD, DEGW), jnp.float32),
        pltpu.VMEM((NCHUNK, CHUNK), jnp.int32),
        pltpu.VMEM((CHUNK, DEGW), jnp.float32),
        pltpu.SemaphoreType.DMA,
        pltpu.SemaphoreType.DMA,
    ],
    compiler_params=pltpu.CompilerParams(use_tc_tiling_on_sc=False),
)


def _prep_body(x_ref, w_ref, b_ref, degp_ref, x0_ref, p0_ref, dv_ref):
    h = jnp.dot(x_ref[...], w_ref[...], preferred_element_type=jnp.float32)
    h = jnp.maximum(h + b_ref[...], 0.0)
    mn = jnp.min(h, axis=1, keepdims=True)
    mx = jnp.max(h, axis=1, keepdims=True)
    x0 = 2.0 * (h - mn) / (mx - mn + 1e-08) - 1.0
    deg = degp_ref[0, :, 0:1] + degp_ref[1, :, 0:1] + 1.0
    dv = jnp.broadcast_to(lax.rsqrt(deg), x0.shape)
    x0_ref[...] = x0
    p0_ref[...] = x0 * dv
    dv_ref[...] = dv


def _layer_body(acc_ref, dv_ref, w_ref, b_ref, x_ref, po_ref):
    dv = dv_ref[...]
    t = (acc_ref[0] + acc_ref[1]) * dv
    xi = jnp.dot(t, w_ref[...], preferred_element_type=jnp.float32) + b_ref[...]
    x_ref[...] = xi
    po_ref[...] = xi * dv


def _final_body(*refs):
    xs = refs[:32]
    w_ref, b_ref, out_ref = refs[32], refs[33], refs[34]

    def mynorm(t):
        mn = jnp.min(t, axis=1, keepdims=True)
        mx = jnp.max(t, axis=1, keepdims=True)
        return 2.0 * (t - mn) / (mx - mn + 1e-08) - 1.0

    vals = [x[...] for x in xs]
    norms = [mynorm(v) for v in vals]
    parts = [vals[0], vals[1]]
    for i in range(2, 32):
        parts.append(norms[i] - norms[i - 2])
    parts[16] = vals[16]
    xx = jnp.concatenate(parts, axis=1)
    out_ref[...] = (
        jnp.dot(xx, w_ref[...], preferred_element_type=jnp.float32)
        + b_ref[...])


def _whole_spec(shape):
    nd = len(shape)
    return pl.BlockSpec(shape, lambda *_: (0,) * nd)


_prep_call = pl.pallas_call(
    _prep_body,
    in_specs=[
        _whole_spec((N_PAD, 128)),
        _whole_spec((128, HF)),
        _whole_spec((1, HF)),
        _whole_spec((NC, N_PAD, DEGW)),
    ],
    out_specs=[_whole_spec((N_PAD, HF))] * 3,
    out_shape=[jax.ShapeDtypeStruct((N_PAD, HF), jnp.float32)] * 3,
)

_layer_call = pl.pallas_call(
    _layer_body,
    in_specs=[
        _whole_spec((NC, N_PAD, HF)),
        _whole_spec((N_PAD, HF)),
        _whole_spec((HF, HF)),
        _whole_spec((1, HF)),
    ],
    out_specs=[_whole_spec((N_PAD, HF))] * 2,
    out_shape=[jax.ShapeDtypeStruct((N_PAD, HF), jnp.float32)] * 2,
)

_RF = 1024
_final_call = pl.pallas_call(
    _final_body,
    grid=(N_PAD // _RF,),
    in_specs=(
        [pl.BlockSpec((_RF, HF), lambda i: (i, 0))] * 32
        + [_whole_spec((32 * HF, 128)), _whole_spec((1, 128))]
    ),
    out_specs=pl.BlockSpec((_RF, 128), lambda i: (i, 0)),
    out_shape=jax.ShapeDtypeStruct((N_PAD, 128), jnp.float32),
)


@jax.jit
def kernel(x, edge_index, W_fc1, b_fc1, Wc, bc, W_out, b_out):
    src = edge_index[0].astype(jnp.int32)
    dst = edge_index[1].astype(jnp.int32)
    e = src.shape[0]
    pad = E_PAD - e
    src3 = jnp.concatenate(
        [src, jnp.zeros((pad,), jnp.int32)]).reshape(NW, NCHUNK, CHUNK)
    dst3 = jnp.concatenate(
        [dst, jnp.full((pad,), N, jnp.int32)]).reshape(NW, NCHUNK, CHUNK)

    xp = jnp.zeros((N_PAD, x.shape[1]), x.dtype).at[:N].set(x)
    zeros32 = jnp.zeros((N_PAD, HF), jnp.float32)
    zeros16 = jnp.zeros((N_PAD, DEGW), jnp.float32)

    degp = _deg_pass(dst3, zeros16)
    x0, p, dv = _prep_call(xp, W_fc1, b_fc1.reshape(1, HF), degp)

    xs = [x0]
    for i in range(31):
        accp = _edge_pass(p, src3, dst3, zeros32)
        xi, p = _layer_call(accp, dv, Wc[i], bc[i].reshape(1, HF))
        xs.append(xi)

    out = _final_call(*xs, W_out, b_out.reshape(1, 128))
    return out[:N]
